# baseline (speedup 1.0000x reference)
import functools

import jax
import jax.numpy as jnp
from jax import lax
from jax.experimental import pallas as pl
from jax.experimental.pallas import tpu as pltpu
from jax.experimental.pallas import tpu_sc as plsc

T = 200
D = 64
B = 4096
V = 100000

NC = 2
NS = 16
NW = NC * NS
BBLK = B // NW
LANES = 16
NBUF = 3


def _body(x_hbm, tok_hbm, pos_hbm, out_hbm, idx_v, pos_v,
          g0, g1, g2, c0, c1, c2, gs0, gs1, gs2, ss0, ss1, ss2):
    gbufs = (g0, g1, g2)
    cbufs = (c0, c1, c2)
    gsems = (gs0, gs1, gs2)
    ssems = (ss0, ss1, ss2)

    wid = lax.axis_index("s") * NC + lax.axis_index("c")

    pltpu.sync_copy(x_hbm.at[:, wid], idx_v)
    pltpu.sync_copy(pos_hbm, pos_v)

    rows_g = [(jax.lax.iota(jnp.int32, LANES) + g * LANES) for g in range(8)]

    def g_start(t, j):
        pltpu.make_async_copy(
            tok_hbm.at[idx_v.at[t // 8, t % 8]], gbufs[j], gsems[j]).start()

    def g_wait(t, j):
        pltpu.make_async_copy(
            tok_hbm.at[idx_v.at[t // 8, t % 8]], gbufs[j], gsems[j]).wait()

    def s_copy(t, j):
        return pltpu.make_async_copy(
            cbufs[j], out_hbm.at[t, :, wid], ssems[j])

    def compute(t, j):
        gbuf = gbufs[j]
        cbuf = cbufs[j]

        tvec = jnp.broadcast_to(jnp.int32(t), (LANES,))

        @plsc.parallel_loop(0, D // 8)
        def _dt(dt):
            for dr in range(8):
                d = dt * 8 + dr
                dvec = jnp.broadcast_to(jnp.int32(d), (LANES,))
                pvec = plsc.load_gather(pos_v, [tvec, dvec])
                for g in range(8):
                    vals = plsc.load_gather(gbuf, [rows_g[g], dvec])
                    cbuf[dt, dr, pl.ds(g * LANES, LANES)] = vals + pvec

    def step(t, j, prefetch, swait):
        if prefetch:
            g_start(t + 2, (j + 2) % NBUF)
        g_wait(t, j)
        if swait:
            s_copy(t - NBUF, j).wait()
        compute(t, j)
        s_copy(t, j).start()

    g_start(0, 0)
    g_start(1, 1)
    step(0, 0, True, False)
    step(1, 1, True, False)
    step(2, 2, True, False)

    @pl.loop(0, (T - 2 * NBUF + 1) // NBUF)
    def _main(i):
        t0 = NBUF + i * NBUF
        for jj in range(NBUF):
            step(t0 + jj, jj, True, True)

    step(T - 2, 0, False, True)
    step(T - 1, 1, False, True)
    for t, j in ((T - 3, 2), (T - 2, 0), (T - 1, 1)):
        s_copy(t, j).wait()


@functools.partial(jax.jit, static_argnames=())
def kernel(x, token_table, pos_table):
    x4 = (x.astype(jnp.int32).T
          .reshape(T // 8, 8, B // 128, 128).transpose(0, 2, 1, 3))
    tok_p = jnp.pad(token_table, ((0, 0), (0, 128 - D)))
    f = pl.kernel(
        _body,
        out_type=jax.ShapeDtypeStruct((T, D // 8, B // 128, 8, 128),
                                      jnp.float32),
        mesh=plsc.VectorSubcoreMesh(core_axis_name="c", subcore_axis_name="s"),
        compiler_params=pltpu.CompilerParams(use_tc_tiling_on_sc=True,
                                             needs_layout_passes=False),
        scratch_types=[
            pltpu.VMEM((T // 8, 8, BBLK), jnp.int32),
            pltpu.VMEM((T, D), jnp.float32),
        ] + [pltpu.VMEM((BBLK, 128), jnp.float32)] * NBUF
          + [pltpu.VMEM((D // 8, 8, BBLK), jnp.float32)] * NBUF
          + [pltpu.SemaphoreType.DMA] * (2 * NBUF),
    )
    out5 = f(x4, tok_p, pos_table)
    return out5.transpose(2, 4, 0, 1, 3).reshape(B, T, D)


# speedup vs baseline: 1.5566x; 1.5566x over previous
import functools

import jax
import jax.numpy as jnp
from jax import lax
from jax.experimental import pallas as pl
from jax.experimental.pallas import tpu as pltpu
from jax.experimental.pallas import tpu_sc as plsc

T = 200
D = 64
B = 4096
V = 100000

NC = 2
NS = 16
NW = NC * NS
LOOK = B * T
PER_W = LOOK // NW
CHUNK = 64
NCHUNK = PER_W // CHUNK
LANES = 16
NBUF = 4
IDXROWS = PER_W // 128


def _body(x_hbm, tok_hbm, pos_hbm, out_hbm, idx_v, pos2_v,
          g0, g1, g2, g3, c0, c1, c2, c3, gs0, gs1, gs2, gs3,
          ss0, ss1, ss2, ss3):
    gbufs = (g0, g1, g2, g3)
    cbufs = (c0, c1, c2, c3)
    gsems = (gs0, gs1, gs2, gs3)
    ssems = (ss0, ss1, ss2, ss3)

    wid = lax.axis_index("s") * NC + lax.axis_index("c")
    row_base = wid * PER_W

    pltpu.sync_copy(x_hbm.at[pl.ds(wid * IDXROWS, IDXROWS), :], idx_v)
    pltpu.sync_copy(pos_hbm, pos2_v.at[pl.ds(0, T), :])
    pltpu.sync_copy(pos_hbm.at[pl.ds(0, CHUNK), :], pos2_v.at[pl.ds(T, CHUNK), :])

    def idx_slice(k, half):
        return idx_v.at[k // 2, pl.ds(half * CHUNK, CHUNK)]

    def g_start(k, half, j):
        pltpu.make_async_copy(
            tok_hbm.at[idx_slice(k, half)], gbufs[j], gsems[j]).start()

    def g_wait(k, half, j):
        pltpu.make_async_copy(
            tok_hbm.at[idx_slice(k, half)], gbufs[j], gsems[j]).wait()

    def s_copy(k, j):
        return pltpu.make_async_copy(
            cbufs[j], out_hbm.at[pl.ds(row_base + k * CHUNK, CHUNK), :],
            ssems[j])

    def process(k, half, j):
        g_wait(k, half, j)
        p0 = lax.rem(k * CHUNK, T)
        gbuf = gbufs[j]
        cbuf = cbufs[j]

        @plsc.parallel_loop(0, CHUNK, unroll=4)
        def _add(r):
            for jj in range(D // LANES):
                s = pl.ds(jj * LANES, LANES)
                cbuf[r, s] = gbuf[r, s] + pos2_v[p0 + r, s]

        s_copy(k, j).start()

    g_start(0, 0, 0)
    g_start(1, 1, 1)
    g_start(2, 0, 2)
    process(0, 0, 0)
    g_start(3, 1, 3)
    process(1, 1, 1)

    @pl.loop(0, (NCHUNK - NBUF) // NBUF)
    def _main(i):
        k = 2 + i * NBUF
        for jj in range(NBUF):
            kk = k + jj
            half = jj % 2
            j_cur = (2 + jj) % NBUF
            j_pre = (j_cur + 2) % NBUF
            s_copy(kk - 2, j_pre).wait()
            g_start(kk + 2, half, j_pre)
            process(kk, half, j_cur)

    process(NCHUNK - 2, 0, 2)
    process(NCHUNK - 1, 1, 3)
    for kk, j in ((NCHUNK - 4, 0), (NCHUNK - 3, 1),
                  (NCHUNK - 2, 2), (NCHUNK - 1, 3)):
        s_copy(kk, j).wait()


@functools.partial(jax.jit, static_argnames=())
def kernel(x, token_table, pos_table):
    x2 = x.astype(jnp.int32).reshape(LOOK // 128, 128)
    tok_p = jnp.pad(token_table, ((0, 0), (0, 128 - D)))
    f = pl.kernel(
        _body,
        out_type=jax.ShapeDtypeStruct((LOOK, D), jnp.float32),
        mesh=plsc.VectorSubcoreMesh(core_axis_name="c", subcore_axis_name="s"),
        compiler_params=pltpu.CompilerParams(use_tc_tiling_on_sc=True),
        scratch_types=[
            pltpu.VMEM((IDXROWS, 128), jnp.int32),
            pltpu.VMEM((T + CHUNK, D), jnp.float32),
        ] + [pltpu.VMEM((CHUNK, 128), jnp.float32)] * NBUF
          + [pltpu.VMEM((CHUNK, D), jnp.float32)] * NBUF
          + [pltpu.SemaphoreType.DMA] * (2 * NBUF),
    )
    out = f(x2, tok_p, pos_table)
    return out.reshape(B, T, D)


# speedup vs baseline: 1.5621x; 1.0036x over previous
import functools

import jax
import jax.numpy as jnp
from jax import lax
from jax.experimental import pallas as pl
from jax.experimental.pallas import tpu as pltpu
from jax.experimental.pallas import tpu_sc as plsc

T = 200
D = 64
B = 4096
V = 100000

NC = 2
NS = 16
NW = NC * NS
LOOK = B * T
PER_W = LOOK // NW
CHUNK = 128
NCHUNK = PER_W // CHUNK
LANES = 16


def _body(x_hbm, tok_hbm, pos_hbm, out_hbm, idx_v, pos2_v,
          g0, g1, c0, c1, gs0, gs1, ss0, ss1):
    gbufs = (g0, g1)
    cbufs = (c0, c1)
    gsems = (gs0, gs1)
    ssems = (ss0, ss1)

    wid = lax.axis_index("s") * NC + lax.axis_index("c")
    row_base = wid * PER_W

    pltpu.sync_copy(x_hbm.at[pl.ds(wid * NCHUNK, NCHUNK), :], idx_v)
    pltpu.sync_copy(pos_hbm, pos2_v)

    def g_start(k, j):
        pltpu.make_async_copy(
            tok_hbm.at[idx_v.at[k]], gbufs[j], gsems[j]).start()

    def g_wait(k, j):
        pltpu.make_async_copy(
            tok_hbm.at[idx_v.at[k]], gbufs[j], gsems[j]).wait()

    def s_copy(k, j):
        return pltpu.make_async_copy(
            cbufs[j], out_hbm.at[pl.ds(row_base + k * CHUNK, CHUNK), :],
            ssems[j])

    def step(kk, j, swait, prefetch):
        g_wait(kk, j)
        if swait:
            s_copy(kk - 2, j).wait()
        p0 = lax.rem(kk * CHUNK, T)
        n1 = jnp.minimum(CHUNK, T - p0)
        gbuf = gbufs[j]
        cbuf = cbufs[j]

        @plsc.parallel_loop(0, n1, unroll=2)
        def _add_lo(r):
            for jj in range(D // LANES):
                s = pl.ds(jj * LANES, LANES)
                cbuf[r, s] = gbuf[r, s] + pos2_v[p0 + r, s]

        @plsc.parallel_loop(n1, CHUNK, unroll=2)
        def _add_hi(r):
            for jj in range(D // LANES):
                s = pl.ds(jj * LANES, LANES)
                cbuf[r, s] = gbuf[r, s] + pos2_v[p0 + r - T, s]

        s_copy(kk, j).start()
        if prefetch:
            g_start(kk + 2, j)

    g_start(0, 0)
    g_start(1, 1)
    step(0, 0, False, True)
    step(1, 1, False, True)

    @pl.loop(0, (NCHUNK - 4) // 2)
    def _main(i):
        k0 = 2 + i * 2
        step(k0, 0, True, True)
        step(k0 + 1, 1, True, True)

    step(NCHUNK - 2, 0, True, False)
    step(NCHUNK - 1, 1, True, False)
    s_copy(NCHUNK - 2, 0).wait()
    s_copy(NCHUNK - 1, 1).wait()


@functools.partial(jax.jit, static_argnames=())
def kernel(x, token_table, pos_table):
    x2 = x.astype(jnp.int32).reshape(LOOK // 128, 128)
    tok_p = jnp.pad(token_table, ((0, 0), (0, 128 - D)))
    f = pl.kernel(
        _body,
        out_type=jax.ShapeDtypeStruct((LOOK, D), jnp.float32),
        mesh=plsc.VectorSubcoreMesh(core_axis_name="c", subcore_axis_name="s"),
        compiler_params=pltpu.CompilerParams(use_tc_tiling_on_sc=True),
        scratch_types=[
            pltpu.VMEM((NCHUNK, 128), jnp.int32),
            pltpu.VMEM((T, D), jnp.float32),
        ] + [pltpu.VMEM((CHUNK, 128), jnp.float32)] * 2
          + [pltpu.VMEM((CHUNK, D), jnp.float32)] * 2
          + [pltpu.SemaphoreType.DMA] * 4,
    )
    out = f(x2, tok_p, pos_table)
    return out.reshape(B, T, D)


# speedup vs baseline: 1.5655x; 1.0021x over previous
import functools

import jax
import jax.numpy as jnp
from jax import lax
from jax.experimental import pallas as pl
from jax.experimental.pallas import tpu as pltpu
from jax.experimental.pallas import tpu_sc as plsc

T = 200
D = 64
B = 4096
V = 100000

NC = 2
NS = 16
NW = NC * NS
LOOK = B * T
PER_W = LOOK // NW
CHUNK = 128
NCHUNK = PER_W // CHUNK
LANES = 16


def _body(x_hbm, tok_hbm, pos_hbm, out_hbm, idx_v, pos2_v,
          g0, g1, c0, c1, gsa0, gsa1, gsb0, gsb1, ss0, ss1):
    gbufs = (g0, g1)
    cbufs = (c0, c1)
    gsemsa = (gsa0, gsa1)
    gsemsb = (gsb0, gsb1)
    ssems = (ss0, ss1)
    HALF = CHUNK // 2

    wid = lax.axis_index("s") * NC + lax.axis_index("c")
    row_base = wid * PER_W

    pltpu.sync_copy(x_hbm.at[pl.ds(wid * NCHUNK, NCHUNK), :], idx_v)
    pltpu.sync_copy(pos_hbm, pos2_v)

    def g_copy(k, half, j):
        sem = (gsemsa, gsemsb)[half][j]
        return pltpu.make_async_copy(
            tok_hbm.at[idx_v.at[k, pl.ds(half * HALF, HALF)]],
            gbufs[j].at[pl.ds(half * HALF, HALF), :], sem)

    def s_copy(k, j):
        return pltpu.make_async_copy(
            cbufs[j], out_hbm.at[pl.ds(row_base + k * CHUNK, CHUNK), :],
            ssems[j])

    def add_rows(gbuf, cbuf, p0, lo, hi, wrapped):
        @plsc.parallel_loop(lo, hi, unroll=2)
        def _add(r):
            pr = p0 + r - T if wrapped else p0 + r
            for jj in range(D // LANES):
                s = pl.ds(jj * LANES, LANES)
                cbuf[r, s] = gbuf[r, s] + pos2_v[pr, s]

    def step(kk, j, swait, prefetch):
        p0 = lax.rem(kk * CHUNK, T)
        n1 = jnp.minimum(CHUNK, T - p0)
        a = jnp.clip(n1, 0, HALF)
        b = jnp.clip(n1, HALF, CHUNK)
        gbuf = gbufs[j]
        cbuf = cbufs[j]

        g_copy(kk, 0, j).wait()
        if swait:
            s_copy(kk - 2, j).wait()
        add_rows(gbuf, cbuf, p0, 0, a, False)
        add_rows(gbuf, cbuf, p0, a, HALF, True)
        if prefetch:
            g_copy(kk + 2, 0, j).start()
        g_copy(kk, 1, j).wait()
        add_rows(gbuf, cbuf, p0, HALF, b, False)
        add_rows(gbuf, cbuf, p0, b, CHUNK, True)
        if prefetch:
            g_copy(kk + 2, 1, j).start()
        s_copy(kk, j).start()

    for k, j in ((0, 0), (1, 1)):
        g_copy(k, 0, j).start()
        g_copy(k, 1, j).start()
    step(0, 0, False, True)
    step(1, 1, False, True)

    @pl.loop(0, (NCHUNK - 4) // 2)
    def _main(i):
        k0 = 2 + i * 2
        step(k0, 0, True, True)
        step(k0 + 1, 1, True, True)

    step(NCHUNK - 2, 0, True, False)
    step(NCHUNK - 1, 1, True, False)
    s_copy(NCHUNK - 2, 0).wait()
    s_copy(NCHUNK - 1, 1).wait()


@functools.partial(jax.jit, static_argnames=())
def kernel(x, token_table, pos_table):
    x2 = x.astype(jnp.int32).reshape(LOOK // 128, 128)
    tok_p = jnp.pad(token_table, ((0, 0), (0, 128 - D)))
    f = pl.kernel(
        _body,
        out_type=jax.ShapeDtypeStruct((LOOK, D), jnp.float32),
        mesh=plsc.VectorSubcoreMesh(core_axis_name="c", subcore_axis_name="s"),
        compiler_params=pltpu.CompilerParams(use_tc_tiling_on_sc=True),
        scratch_types=[
            pltpu.VMEM((NCHUNK, 128), jnp.int32),
            pltpu.VMEM((T, D), jnp.float32),
        ] + [pltpu.VMEM((CHUNK, 128), jnp.float32)] * 2
          + [pltpu.VMEM((CHUNK, D), jnp.float32)] * 2
          + [pltpu.SemaphoreType.DMA] * 6,
    )
    out = f(x2, tok_p, pos_table)
    return out.reshape(B, T, D)
